# Initial kernel scaffold; baseline (speedup 1.0000x reference)
import jax
import jax.numpy as jnp
from jax.experimental import pallas as pl


def kernel(x, edge_index, W1, b1, W2, b2):
    raise NotImplementedError("write your pallas kernel here")


# speedup vs baseline: 31.6913x; 31.6913x over previous
import functools

import jax
import jax.numpy as jnp
from jax import lax
from jax.experimental import pallas as pl
from jax.experimental.pallas import tpu as pltpu
from jax.experimental.pallas import tpu_sc as plsc

N = 10000
NP = 10240
E = 320000
F_IN = 128
H = 16
WIN = 128
WPT = 80
NWRK = 32
EP = NWRK * WPT * WIN
GROUP = 8
NGROUP = WPT // GROUP
RPT = NP // 16

_mesh = plsc.VectorSubcoreMesh(core_axis_name="c", subcore_axis_name="s")
_f32 = jnp.float32


def _zero_rows(rows, n):
    @pl.loop(0, n)
    def _z(i):
        rows[i] = jnp.zeros((16,), _f32)


def _agg_body(y_hbm, srcr_hbm, dstr_hbm, out_hbm, sidx, didx, rows, acc, sem):
    c = lax.axis_index("c")
    s = lax.axis_index("s")
    wid = c * 16 + s
    pltpu.sync_copy(srcr_hbm.at[pl.ds(wid * WPT, WPT)], sidx)
    pltpu.sync_copy(dstr_hbm.at[pl.ds(wid * WPT, WPT)], didx)
    _zero_rows(rows, RPT)
    pltpu.sync_copy(rows.at[pl.ds(0, RPT)], acc.at[pl.ds(s * RPT, RPT)])
    plsc.subcore_barrier()

    @pl.loop(0, NGROUP)
    def _group(g):
        descs = []
        for j in range(GROUP):
            w = g * GROUP + j
            descs.append(
                pltpu.async_copy(y_hbm.at[sidx.at[w]],
                                 rows.at[pl.ds(j * WIN, WIN)], sem))
        for d in descs:
            d.wait()
        for j in range(GROUP):
            w = g * GROUP + j
            pltpu.sync_copy(rows.at[pl.ds(j * WIN, WIN)],
                            acc.at[didx.at[w]], add=True)

    plsc.subcore_barrier()
    pltpu.sync_copy(acc.at[pl.ds(s * RPT, RPT)],
                    out_hbm.at[c, pl.ds(s * RPT, RPT)])


_agg = functools.partial(
    pl.kernel,
    out_type=jax.ShapeDtypeStruct((2, NP, H), _f32),
    mesh=_mesh,
    compiler_params=pltpu.CompilerParams(use_tc_tiling_on_sc=False),
    scratch_types=[
        pltpu.VMEM((WPT, WIN), jnp.int32),
        pltpu.VMEM((WPT, WIN), jnp.int32),
        pltpu.VMEM((GROUP * WIN, H), _f32),
        pltpu.VMEM_SHARED((NP, H), _f32),
        pltpu.SemaphoreType.DMA,
    ],
)(_agg_body)


def _deg_body(dstr_hbm, out_hbm, didx, rows, acc):
    c = lax.axis_index("c")
    s = lax.axis_index("s")
    wid = c * 16 + s
    pltpu.sync_copy(dstr_hbm.at[pl.ds(wid * WPT, WPT)], didx)
    _zero_rows(rows, RPT)
    pltpu.sync_copy(rows.at[pl.ds(0, RPT)], acc.at[pl.ds(s * RPT, RPT)])

    @pl.loop(0, WIN)
    def _ones(i):
        rows[i] = jnp.ones((16,), _f32)

    plsc.subcore_barrier()

    @pl.loop(0, WPT)
    def _w(w):
        pltpu.sync_copy(rows.at[pl.ds(0, WIN)], acc.at[didx.at[w]], add=True)

    plsc.subcore_barrier()
    pltpu.sync_copy(acc.at[pl.ds(s * RPT, RPT)],
                    out_hbm.at[c, pl.ds(s * RPT, RPT)])


_deg = functools.partial(
    pl.kernel,
    out_type=jax.ShapeDtypeStruct((2, NP, H), _f32),
    mesh=_mesh,
    compiler_params=pltpu.CompilerParams(use_tc_tiling_on_sc=False),
    scratch_types=[
        pltpu.VMEM((WPT, WIN), jnp.int32),
        pltpu.VMEM((RPT, H), _f32),
        pltpu.VMEM_SHARED((NP, H), _f32),
    ],
)(_deg_body)


_BLK = 1024
_NBLK = NP // _BLK


def _tc1_body(x_ref, w1_ref, p0_ref, p1_ref, y_ref, dinv_ref):
    deg = p0_ref[...] + p1_ref[...] + 1.0
    dinv = lax.rsqrt(deg)
    xw = jnp.dot(x_ref[...], w1_ref[...], preferred_element_type=_f32)
    y_ref[...] = xw * dinv
    dinv_ref[...] = dinv


_tc1 = pl.pallas_call(
    _tc1_body,
    grid=(_NBLK,),
    in_specs=[
        pl.BlockSpec((_BLK, F_IN), lambda i: (i, 0)),
        pl.BlockSpec((F_IN, H), lambda i: (0, 0)),
        pl.BlockSpec((_BLK, H), lambda i: (i, 0)),
        pl.BlockSpec((_BLK, H), lambda i: (i, 0)),
    ],
    out_specs=[
        pl.BlockSpec((_BLK, H), lambda i: (i, 0)),
        pl.BlockSpec((_BLK, H), lambda i: (i, 0)),
    ],
    out_shape=[
        jax.ShapeDtypeStruct((NP, H), _f32),
        jax.ShapeDtypeStruct((NP, H), _f32),
    ],
)


def _tc2_body(y1_ref, q0_ref, q1_ref, dinv_ref, b1_ref, w2_ref, y2_ref):
    i = pl.program_id(0)
    acc = y1_ref[...] + q0_ref[...] + q1_ref[...]
    h = jnp.maximum(acc * dinv_ref[...] + b1_ref[...], 0.0)
    row = i * _BLK + lax.broadcasted_iota(jnp.int32, (_BLK, H), 0)
    h = jnp.where(row < N, h, 0.0)
    y2 = jnp.dot(h, w2_ref[...], preferred_element_type=_f32)
    y2_ref[...] = y2 * dinv_ref[...]


_tc2 = pl.pallas_call(
    _tc2_body,
    grid=(_NBLK,),
    in_specs=[
        pl.BlockSpec((_BLK, H), lambda i: (i, 0)),
        pl.BlockSpec((_BLK, H), lambda i: (i, 0)),
        pl.BlockSpec((_BLK, H), lambda i: (i, 0)),
        pl.BlockSpec((_BLK, H), lambda i: (i, 0)),
        pl.BlockSpec((1, H), lambda i: (0, 0)),
        pl.BlockSpec((H, H), lambda i: (0, 0)),
    ],
    out_specs=pl.BlockSpec((_BLK, H), lambda i: (i, 0)),
    out_shape=jax.ShapeDtypeStruct((NP, H), _f32),
)


def _tc3_body(y2_ref, r0_ref, r1_ref, dinv_ref, b2_ref, out_ref):
    acc = y2_ref[...] + r0_ref[...] + r1_ref[...]
    out_ref[...] = jnp.maximum(acc * dinv_ref[...] + b2_ref[...], 0.0)


_tc3 = pl.pallas_call(
    _tc3_body,
    grid=(_NBLK,),
    in_specs=[
        pl.BlockSpec((_BLK, H), lambda i: (i, 0)),
        pl.BlockSpec((_BLK, H), lambda i: (i, 0)),
        pl.BlockSpec((_BLK, H), lambda i: (i, 0)),
        pl.BlockSpec((_BLK, H), lambda i: (i, 0)),
        pl.BlockSpec((1, H), lambda i: (0, 0)),
    ],
    out_specs=pl.BlockSpec((_BLK, H), lambda i: (i, 0)),
    out_shape=jax.ShapeDtypeStruct((NP, H), _f32),
)


def kernel(x, edge_index, W1, b1, W2, b2):
    src = edge_index[0]
    dst = edge_index[1]
    pad = jnp.full((EP - E,), N, dtype=jnp.int32)
    srcr = jnp.concatenate([src, pad]).reshape(NWRK * WPT, WIN)
    dstr = jnp.concatenate([dst, pad]).reshape(NWRK * WPT, WIN)
    xp = jnp.pad(x, ((0, NP - N), (0, 0)))

    degp = _deg(dstr)
    y1, dinv = _tc1(xp, W1, degp[0], degp[1])
    q = _agg(y1, srcr, dstr)
    y2 = _tc2(y1, q[0], q[1], dinv, b1.reshape(1, H), W2)
    r = _agg(y2, srcr, dstr)
    outp = _tc3(y2, r[0], r[1], dinv, b2.reshape(1, H))
    return outp[:N]


# speedup vs baseline: 33.7369x; 1.0645x over previous
import functools

import jax
import jax.numpy as jnp
from jax import lax
from jax.experimental import pallas as pl
from jax.experimental.pallas import tpu as pltpu
from jax.experimental.pallas import tpu_sc as plsc

N = 10000
NP = 10240
E = 320000
F_IN = 128
H = 16
WIN = 128
WPT = 80
NWRK = 32
EP = NWRK * WPT * WIN
GROUP = 8
NGROUP = WPT // GROUP
RPT = NP // 16

_mesh = plsc.VectorSubcoreMesh(core_axis_name="c", subcore_axis_name="s")
_f32 = jnp.float32


def _zero_rows(rows, n):
    @pl.loop(0, n)
    def _z(i):
        rows[i] = jnp.zeros((16,), _f32)


_GW = GROUP * WIN


def _agg_body(y_hbm, srcr_hbm, dstr_hbm, out_hbm, sidx, didx, rows, acc,
              sem_g, sem_s):
    c = lax.axis_index("c")
    s = lax.axis_index("s")
    wid = c * 16 + s
    pltpu.sync_copy(srcr_hbm.at[pl.ds(wid * WPT, WPT)], sidx)
    pltpu.sync_copy(dstr_hbm.at[pl.ds(wid * WPT, WPT)], didx)
    _zero_rows(rows, RPT)
    pltpu.sync_copy(rows.at[pl.ds(0, RPT)], acc.at[pl.ds(s * RPT, RPT)])
    plsc.subcore_barrier()

    def fire_gathers(g, off):
        for j in range(GROUP):
            pltpu.async_copy(y_hbm.at[sidx.at[g * GROUP + j]],
                             rows.at[pl.ds(off + j * WIN, WIN)], sem_g)

    def fire_scatters(g, off):
        for j in range(GROUP):
            pltpu.async_copy(rows.at[pl.ds(off + j * WIN, WIN)],
                             acc.at[didx.at[g * GROUP + j]], sem_s, add=True)

    def drain(sem):
        for _ in range(GROUP):
            pltpu.make_async_copy(y_hbm.at[pl.ds(0, WIN)],
                                  rows.at[pl.ds(0, WIN)], sem).wait()

    fire_gathers(0, 0)

    @pl.loop(0, NGROUP)
    def _group(g):
        off = (g % 2) * _GW
        noff = _GW - off

        @pl.when(g > 0)
        def _():
            drain(sem_s)

        drain(sem_g)

        @pl.when(g + 1 < NGROUP)
        def _():
            fire_gathers(g + 1, noff)

        fire_scatters(g, off)

    drain(sem_s)
    plsc.subcore_barrier()
    pltpu.sync_copy(acc.at[pl.ds(s * RPT, RPT)],
                    out_hbm.at[c, pl.ds(s * RPT, RPT)])


_agg = functools.partial(
    pl.kernel,
    out_type=jax.ShapeDtypeStruct((2, NP, H), _f32),
    mesh=_mesh,
    compiler_params=pltpu.CompilerParams(use_tc_tiling_on_sc=False),
    scratch_types=[
        pltpu.VMEM((WPT, WIN), jnp.int32),
        pltpu.VMEM((WPT, WIN), jnp.int32),
        pltpu.VMEM((2 * _GW, H), _f32),
        pltpu.VMEM_SHARED((NP, H), _f32),
        pltpu.SemaphoreType.DMA,
        pltpu.SemaphoreType.DMA,
    ],
)(_agg_body)


def _deg_body(dstr_hbm, out_hbm, didx, rows, acc, sem_s):
    c = lax.axis_index("c")
    s = lax.axis_index("s")
    wid = c * 16 + s
    pltpu.sync_copy(dstr_hbm.at[pl.ds(wid * WPT, WPT)], didx)
    _zero_rows(rows, RPT)
    pltpu.sync_copy(rows.at[pl.ds(0, RPT)], acc.at[pl.ds(s * RPT, RPT)])

    @pl.loop(0, WIN)
    def _ones(i):
        rows[i] = jnp.ones((16,), _f32)

    plsc.subcore_barrier()

    @pl.loop(0, NGROUP)
    def _group(g):
        for j in range(GROUP):
            pltpu.async_copy(rows.at[pl.ds(0, WIN)],
                             acc.at[didx.at[g * GROUP + j]], sem_s, add=True)
        for _ in range(GROUP):
            pltpu.make_async_copy(rows.at[pl.ds(0, WIN)],
                                  acc.at[pl.ds(0, WIN)], sem_s).wait()

    plsc.subcore_barrier()
    pltpu.sync_copy(acc.at[pl.ds(s * RPT, RPT)],
                    out_hbm.at[c, pl.ds(s * RPT, RPT)])


_deg = functools.partial(
    pl.kernel,
    out_type=jax.ShapeDtypeStruct((2, NP, H), _f32),
    mesh=_mesh,
    compiler_params=pltpu.CompilerParams(use_tc_tiling_on_sc=False),
    scratch_types=[
        pltpu.VMEM((WPT, WIN), jnp.int32),
        pltpu.VMEM((RPT, H), _f32),
        pltpu.VMEM_SHARED((NP, H), _f32),
        pltpu.SemaphoreType.DMA,
    ],
)(_deg_body)


_BLK = 1024
_NBLK = NP // _BLK


def _tc1_body(x_ref, w1_ref, p0_ref, p1_ref, y_ref, dinv_ref):
    deg = p0_ref[...] + p1_ref[...] + 1.0
    dinv = lax.rsqrt(deg)
    xw = jnp.dot(x_ref[...], w1_ref[...], preferred_element_type=_f32)
    y_ref[...] = xw * dinv
    dinv_ref[...] = dinv


_tc1 = pl.pallas_call(
    _tc1_body,
    grid=(_NBLK,),
    in_specs=[
        pl.BlockSpec((_BLK, F_IN), lambda i: (i, 0)),
        pl.BlockSpec((F_IN, H), lambda i: (0, 0)),
        pl.BlockSpec((_BLK, H), lambda i: (i, 0)),
        pl.BlockSpec((_BLK, H), lambda i: (i, 0)),
    ],
    out_specs=[
        pl.BlockSpec((_BLK, H), lambda i: (i, 0)),
        pl.BlockSpec((_BLK, H), lambda i: (i, 0)),
    ],
    out_shape=[
        jax.ShapeDtypeStruct((NP, H), _f32),
        jax.ShapeDtypeStruct((NP, H), _f32),
    ],
)


def _tc2_body(y1_ref, q0_ref, q1_ref, dinv_ref, b1_ref, w2_ref, y2_ref):
    i = pl.program_id(0)
    acc = y1_ref[...] + q0_ref[...] + q1_ref[...]
    h = jnp.maximum(acc * dinv_ref[...] + b1_ref[...], 0.0)
    row = i * _BLK + lax.broadcasted_iota(jnp.int32, (_BLK, H), 0)
    h = jnp.where(row < N, h, 0.0)
    y2 = jnp.dot(h, w2_ref[...], preferred_element_type=_f32)
    y2_ref[...] = y2 * dinv_ref[...]


_tc2 = pl.pallas_call(
    _tc2_body,
    grid=(_NBLK,),
    in_specs=[
        pl.BlockSpec((_BLK, H), lambda i: (i, 0)),
        pl.BlockSpec((_BLK, H), lambda i: (i, 0)),
        pl.BlockSpec((_BLK, H), lambda i: (i, 0)),
        pl.BlockSpec((_BLK, H), lambda i: (i, 0)),
        pl.BlockSpec((1, H), lambda i: (0, 0)),
        pl.BlockSpec((H, H), lambda i: (0, 0)),
    ],
    out_specs=pl.BlockSpec((_BLK, H), lambda i: (i, 0)),
    out_shape=jax.ShapeDtypeStruct((NP, H), _f32),
)


def _tc3_body(y2_ref, r0_ref, r1_ref, dinv_ref, b2_ref, out_ref):
    acc = y2_ref[...] + r0_ref[...] + r1_ref[...]
    out_ref[...] = jnp.maximum(acc * dinv_ref[...] + b2_ref[...], 0.0)


_tc3 = pl.pallas_call(
    _tc3_body,
    grid=(_NBLK,),
    in_specs=[
        pl.BlockSpec((_BLK, H), lambda i: (i, 0)),
        pl.BlockSpec((_BLK, H), lambda i: (i, 0)),
        pl.BlockSpec((_BLK, H), lambda i: (i, 0)),
        pl.BlockSpec((_BLK, H), lambda i: (i, 0)),
        pl.BlockSpec((1, H), lambda i: (0, 0)),
    ],
    out_specs=pl.BlockSpec((_BLK, H), lambda i: (i, 0)),
    out_shape=jax.ShapeDtypeStruct((NP, H), _f32),
)


def kernel(x, edge_index, W1, b1, W2, b2):
    src = edge_index[0]
    dst = edge_index[1]
    pad = jnp.full((EP - E,), N, dtype=jnp.int32)
    srcr = jnp.concatenate([src, pad]).reshape(NWRK * WPT, WIN)
    dstr = jnp.concatenate([dst, pad]).reshape(NWRK * WPT, WIN)
    xp = jnp.pad(x, ((0, NP - N), (0, 0)))

    degp = _deg(dstr)
    y1, dinv = _tc1(xp, W1, degp[0], degp[1])
    q = _agg(y1, srcr, dstr)
    y2 = _tc2(y1, q[0], q[1], dinv, b1.reshape(1, H), W2)
    r = _agg(y2, srcr, dstr)
    outp = _tc3(y2, r[0], r[1], dinv, b2.reshape(1, H))
    return outp[:N]


# speedup vs baseline: 37.9926x; 1.1261x over previous
import functools

import jax
import jax.numpy as jnp
from jax import lax
from jax.experimental import pallas as pl
from jax.experimental.pallas import tpu as pltpu
from jax.experimental.pallas import tpu_sc as plsc

N = 10000
NP = 10240
E = 320000
F_IN = 128
H = 16
WIN = 128
NWIN = 2560
NWIN_PAD = 2624
EP = NWIN_PAD * WIN
GROUP = 8
AGG_W0, AGG_W1 = 112, 48
DEG_W0, DEG_W1 = 96, 64
RPT = NP // 16

_mesh = plsc.VectorSubcoreMesh(core_axis_name="c", subcore_axis_name="s")
_f32 = jnp.float32


def _zero_rows(rows, n):
    @pl.loop(0, n)
    def _z(i):
        rows[i] = jnp.zeros((16,), _f32)


_GW = GROUP * WIN


def _agg_body(y_hbm, srcr_hbm, dstr_hbm, out_hbm, sidx, didx, rows, acc,
              sem_g, sem_s):
    c = lax.axis_index("c")
    s = lax.axis_index("s")
    wbase = jnp.where(c == 0, s * AGG_W0, 16 * AGG_W0 + s * AGG_W1)
    nwin = jnp.where(c == 0, AGG_W0, AGG_W1)
    ngroup = nwin // GROUP
    pltpu.sync_copy(srcr_hbm.at[pl.ds(wbase, AGG_W0)], sidx)
    pltpu.sync_copy(dstr_hbm.at[pl.ds(wbase, AGG_W0)], didx)
    _zero_rows(rows, RPT)
    pltpu.sync_copy(rows.at[pl.ds(0, RPT)], acc.at[pl.ds(s * RPT, RPT)])
    plsc.subcore_barrier()

    def fire_gathers(g, off):
        for j in range(GROUP):
            pltpu.async_copy(y_hbm.at[sidx.at[g * GROUP + j]],
                             rows.at[pl.ds(off + j * WIN, WIN)], sem_g)

    def fire_scatters(g, off):
        for j in range(GROUP):
            pltpu.async_copy(rows.at[pl.ds(off + j * WIN, WIN)],
                             acc.at[didx.at[g * GROUP + j]], sem_s, add=True)

    def drain(sem):
        for _ in range(GROUP):
            pltpu.make_async_copy(y_hbm.at[pl.ds(0, WIN)],
                                  rows.at[pl.ds(0, WIN)], sem).wait()

    fire_gathers(0, 0)

    @pl.loop(0, ngroup)
    def _group(g):
        off = (g % 2) * _GW
        noff = _GW - off

        @pl.when(g > 0)
        def _():
            drain(sem_s)

        drain(sem_g)

        @pl.when(g + 1 < ngroup)
        def _():
            fire_gathers(g + 1, noff)

        fire_scatters(g, off)

    drain(sem_s)
    plsc.subcore_barrier()
    pltpu.sync_copy(acc.at[pl.ds(s * RPT, RPT)],
                    out_hbm.at[c, pl.ds(s * RPT, RPT)])


_agg = functools.partial(
    pl.kernel,
    out_type=jax.ShapeDtypeStruct((2, NP, H), _f32),
    mesh=_mesh,
    compiler_params=pltpu.CompilerParams(use_tc_tiling_on_sc=False),
    scratch_types=[
        pltpu.VMEM((AGG_W0, WIN), jnp.int32),
        pltpu.VMEM((AGG_W0, WIN), jnp.int32),
        pltpu.VMEM((2 * _GW, H), _f32),
        pltpu.VMEM_SHARED((NP, H), _f32),
        pltpu.SemaphoreType.DMA,
        pltpu.SemaphoreType.DMA,
    ],
)(_agg_body)


def _deg_body(dstr_hbm, out_hbm, didx, rows, acc, sem_s):
    c = lax.axis_index("c")
    s = lax.axis_index("s")
    wbase = jnp.where(c == 0, s * DEG_W0, 16 * DEG_W0 + s * DEG_W1)
    ngroup = jnp.where(c == 0, DEG_W0, DEG_W1) // GROUP
    pltpu.sync_copy(dstr_hbm.at[pl.ds(wbase, DEG_W0)], didx)
    _zero_rows(rows, RPT)
    pltpu.sync_copy(rows.at[pl.ds(0, RPT)], acc.at[pl.ds(s * RPT, RPT)])

    @pl.loop(0, WIN)
    def _ones(i):
        rows[i] = jnp.ones((16,), _f32)

    plsc.subcore_barrier()

    @pl.loop(0, ngroup)
    def _group(g):
        for j in range(GROUP):
            pltpu.async_copy(rows.at[pl.ds(0, WIN)],
                             acc.at[didx.at[g * GROUP + j]], sem_s, add=True)
        for _ in range(GROUP):
            pltpu.make_async_copy(rows.at[pl.ds(0, WIN)],
                                  acc.at[pl.ds(0, WIN)], sem_s).wait()

    plsc.subcore_barrier()
    pltpu.sync_copy(acc.at[pl.ds(s * RPT, RPT)],
                    out_hbm.at[c, pl.ds(s * RPT, RPT)])


_deg = functools.partial(
    pl.kernel,
    out_type=jax.ShapeDtypeStruct((2, NP, H), _f32),
    mesh=_mesh,
    compiler_params=pltpu.CompilerParams(use_tc_tiling_on_sc=False),
    scratch_types=[
        pltpu.VMEM((DEG_W0, WIN), jnp.int32),
        pltpu.VMEM((RPT, H), _f32),
        pltpu.VMEM_SHARED((NP, H), _f32),
        pltpu.SemaphoreType.DMA,
    ],
)(_deg_body)


_BLK = 1024
_NBLK = NP // _BLK


def _tc1_body(x_ref, w1_ref, p_ref, y_ref, dinv_ref):
    p = p_ref[...]
    deg = p[0] + p[1] + 1.0
    dinv = lax.rsqrt(deg)
    xw = jnp.dot(x_ref[...], w1_ref[...], preferred_element_type=_f32)
    y_ref[...] = xw * dinv
    dinv_ref[...] = dinv


_tc1 = pl.pallas_call(
    _tc1_body,
    grid=(_NBLK,),
    in_specs=[
        pl.BlockSpec((_BLK, F_IN), lambda i: (i, 0)),
        pl.BlockSpec((F_IN, H), lambda i: (0, 0)),
        pl.BlockSpec((2, _BLK, H), lambda i: (0, i, 0)),
    ],
    out_specs=[
        pl.BlockSpec((_BLK, H), lambda i: (i, 0)),
        pl.BlockSpec((_BLK, H), lambda i: (i, 0)),
    ],
    out_shape=[
        jax.ShapeDtypeStruct((NP, H), _f32),
        jax.ShapeDtypeStruct((NP, H), _f32),
    ],
)


def _tc2_body(y1_ref, q_ref, dinv_ref, b1_ref, w2_ref, y2_ref):
    i = pl.program_id(0)
    q = q_ref[...]
    acc = y1_ref[...] + q[0] + q[1]
    h = jnp.maximum(acc * dinv_ref[...] + b1_ref[...], 0.0)
    row = i * _BLK + lax.broadcasted_iota(jnp.int32, (_BLK, H), 0)
    h = jnp.where(row < N, h, 0.0)
    y2 = jnp.dot(h, w2_ref[...], preferred_element_type=_f32)
    y2_ref[...] = y2 * dinv_ref[...]


_tc2 = pl.pallas_call(
    _tc2_body,
    grid=(_NBLK,),
    in_specs=[
        pl.BlockSpec((_BLK, H), lambda i: (i, 0)),
        pl.BlockSpec((2, _BLK, H), lambda i: (0, i, 0)),
        pl.BlockSpec((_BLK, H), lambda i: (i, 0)),
        pl.BlockSpec((1, H), lambda i: (0, 0)),
        pl.BlockSpec((H, H), lambda i: (0, 0)),
    ],
    out_specs=pl.BlockSpec((_BLK, H), lambda i: (i, 0)),
    out_shape=jax.ShapeDtypeStruct((NP, H), _f32),
)


def _tc3_body(y2_ref, r_ref, dinv_ref, b2_ref, out_ref):
    r = r_ref[...]
    acc = y2_ref[...] + r[0] + r[1]
    out_ref[...] = jnp.maximum(acc * dinv_ref[...] + b2_ref[...], 0.0)


_tc3 = pl.pallas_call(
    _tc3_body,
    grid=(_NBLK,),
    in_specs=[
        pl.BlockSpec((_BLK, H), lambda i: (i, 0)),
        pl.BlockSpec((2, _BLK, H), lambda i: (0, i, 0)),
        pl.BlockSpec((_BLK, H), lambda i: (i, 0)),
        pl.BlockSpec((1, H), lambda i: (0, 0)),
    ],
    out_specs=pl.BlockSpec((_BLK, H), lambda i: (i, 0)),
    out_shape=jax.ShapeDtypeStruct((NP, H), _f32),
)


def kernel(x, edge_index, W1, b1, W2, b2):
    src = edge_index[0]
    dst = edge_index[1]
    pad = jnp.full((EP - E,), N, dtype=jnp.int32)
    srcr = jnp.concatenate([src, pad]).reshape(NWIN_PAD, WIN)
    dstr = jnp.concatenate([dst, pad]).reshape(NWIN_PAD, WIN)
    xp = jnp.pad(x, ((0, NP - N), (0, 0)))

    degp = _deg(dstr)
    y1, dinv = _tc1(xp, W1, degp)
    q = _agg(y1, srcr, dstr)
    y2 = _tc2(y1, q, dinv, b1.reshape(1, H), W2)
    r = _agg(y2, srcr, dstr)
    outp = _tc3(y2, r, dinv, b2.reshape(1, H))
    return outp[:N]


# speedup vs baseline: 38.6123x; 1.0163x over previous
import functools

import jax
import jax.numpy as jnp
from jax import lax
from jax.experimental import pallas as pl
from jax.experimental.pallas import tpu as pltpu
from jax.experimental.pallas import tpu_sc as plsc

N = 10000
NP = 10240
E = 320000
F_IN = 128
H = 16
WIN = 128
NWIN = 2560
NWIN_PAD = 2624
EP = NWIN_PAD * WIN
GROUP = 8
NBUF = 4
AGG_W0, AGG_W1 = 112, 48
DEG_W0, DEG_W1 = 96, 64
RPT = NP // 16

_mesh = plsc.VectorSubcoreMesh(core_axis_name="c", subcore_axis_name="s")
_f32 = jnp.float32


def _zero_rows(rows, n):
    @pl.loop(0, n)
    def _z(i):
        rows[i] = jnp.zeros((16,), _f32)


_GW = GROUP * WIN


def _agg_body(y_hbm, srcr_hbm, dstr_hbm, out_hbm, sidx, didx, rows, acc,
              sem_g, sem_s):
    c = lax.axis_index("c")
    s = lax.axis_index("s")
    wbase = jnp.where(c == 0, s * AGG_W0, 16 * AGG_W0 + s * AGG_W1)
    nwin = jnp.where(c == 0, AGG_W0, AGG_W1)
    ngroup = nwin // GROUP
    pltpu.sync_copy(srcr_hbm.at[pl.ds(wbase, AGG_W0)], sidx)
    pltpu.sync_copy(dstr_hbm.at[pl.ds(wbase, AGG_W0)], didx)
    _zero_rows(rows, RPT)
    pltpu.sync_copy(rows.at[pl.ds(0, RPT)], acc.at[pl.ds(s * RPT, RPT)])
    plsc.subcore_barrier()

    def fire_gathers(g, slot):
        off = slot * _GW
        for j in range(GROUP):
            pltpu.async_copy(y_hbm.at[sidx.at[g * GROUP + j]],
                             rows.at[pl.ds(off + j * WIN, WIN)],
                             sem_g.at[slot])

    def fire_scatters(g, slot):
        off = slot * _GW
        for j in range(GROUP):
            pltpu.async_copy(rows.at[pl.ds(off + j * WIN, WIN)],
                             acc.at[didx.at[g * GROUP + j]],
                             sem_s.at[slot], add=True)

    def drain(sem_ref, slot):
        for _ in range(GROUP):
            pltpu.make_async_copy(y_hbm.at[pl.ds(0, WIN)],
                                  rows.at[pl.ds(0, WIN)],
                                  sem_ref.at[slot]).wait()

    for g0 in range(NBUF - 1):
        fire_gathers(g0, g0)

    @pl.loop(0, ngroup)
    def _group(g):
        slot = g % NBUF
        drain(sem_g, slot)
        fire_scatters(g, slot)
        nxt = g + NBUF - 1
        nslot = nxt % NBUF

        @pl.when(nxt < ngroup)
        def _():
            @pl.when(g > 0)
            def _():
                drain(sem_s, nslot)
            fire_gathers(nxt, nslot)

    for k in range(NBUF):
        drain(sem_s, k)
    plsc.subcore_barrier()
    pltpu.sync_copy(acc.at[pl.ds(s * RPT, RPT)],
                    out_hbm.at[c, pl.ds(s * RPT, RPT)])


_agg = functools.partial(
    pl.kernel,
    out_type=jax.ShapeDtypeStruct((2, NP, H), _f32),
    mesh=_mesh,
    compiler_params=pltpu.CompilerParams(use_tc_tiling_on_sc=False),
    scratch_types=[
        pltpu.VMEM((AGG_W0, WIN), jnp.int32),
        pltpu.VMEM((AGG_W0, WIN), jnp.int32),
        pltpu.VMEM((NBUF * _GW, H), _f32),
        pltpu.VMEM_SHARED((NP, H), _f32),
        pltpu.SemaphoreType.DMA((NBUF,)),
        pltpu.SemaphoreType.DMA((NBUF,)),
    ],
)(_agg_body)


def _deg_body(dstr_hbm, out_hbm, didx, rows, acc, sem_s):
    c = lax.axis_index("c")
    s = lax.axis_index("s")
    wbase = jnp.where(c == 0, s * DEG_W0, 16 * DEG_W0 + s * DEG_W1)
    ngroup = jnp.where(c == 0, DEG_W0, DEG_W1) // GROUP
    pltpu.sync_copy(dstr_hbm.at[pl.ds(wbase, DEG_W0)], didx)
    _zero_rows(rows, RPT)
    pltpu.sync_copy(rows.at[pl.ds(0, RPT)], acc.at[pl.ds(s * RPT, RPT)])

    @pl.loop(0, WIN)
    def _ones(i):
        rows[i] = jnp.ones((16,), _f32)

    plsc.subcore_barrier()

    @pl.loop(0, ngroup)
    def _group(g):
        for j in range(GROUP):
            pltpu.async_copy(rows.at[pl.ds(0, WIN)],
                             acc.at[didx.at[g * GROUP + j]], sem_s, add=True)
        for _ in range(GROUP):
            pltpu.make_async_copy(rows.at[pl.ds(0, WIN)],
                                  acc.at[pl.ds(0, WIN)], sem_s).wait()

    plsc.subcore_barrier()
    pltpu.sync_copy(acc.at[pl.ds(s * RPT, RPT)],
                    out_hbm.at[c, pl.ds(s * RPT, RPT)])


_deg = functools.partial(
    pl.kernel,
    out_type=jax.ShapeDtypeStruct((2, NP, H), _f32),
    mesh=_mesh,
    compiler_params=pltpu.CompilerParams(use_tc_tiling_on_sc=False),
    scratch_types=[
        pltpu.VMEM((DEG_W0, WIN), jnp.int32),
        pltpu.VMEM((RPT, H), _f32),
        pltpu.VMEM_SHARED((NP, H), _f32),
        pltpu.SemaphoreType.DMA,
    ],
)(_deg_body)


_BLK = 1024
_NBLK = NP // _BLK


def _tc1_body(x_ref, w1_ref, p_ref, y_ref, dinv_ref):
    p = p_ref[...]
    deg = p[0] + p[1] + 1.0
    dinv = lax.rsqrt(deg)
    xw = jnp.dot(x_ref[...], w1_ref[...], preferred_element_type=_f32)
    y_ref[...] = xw * dinv
    dinv_ref[...] = dinv


_tc1 = pl.pallas_call(
    _tc1_body,
    grid=(_NBLK,),
    in_specs=[
        pl.BlockSpec((_BLK, F_IN), lambda i: (i, 0)),
        pl.BlockSpec((F_IN, H), lambda i: (0, 0)),
        pl.BlockSpec((2, _BLK, H), lambda i: (0, i, 0)),
    ],
    out_specs=[
        pl.BlockSpec((_BLK, H), lambda i: (i, 0)),
        pl.BlockSpec((_BLK, H), lambda i: (i, 0)),
    ],
    out_shape=[
        jax.ShapeDtypeStruct((NP, H), _f32),
        jax.ShapeDtypeStruct((NP, H), _f32),
    ],
)


def _tc2_body(y1_ref, q_ref, dinv_ref, b1_ref, w2_ref, y2_ref):
    i = pl.program_id(0)
    q = q_ref[...]
    acc = y1_ref[...] + q[0] + q[1]
    h = jnp.maximum(acc * dinv_ref[...] + b1_ref[...], 0.0)
    row = i * _BLK + lax.broadcasted_iota(jnp.int32, (_BLK, H), 0)
    h = jnp.where(row < N, h, 0.0)
    y2 = jnp.dot(h, w2_ref[...], preferred_element_type=_f32)
    y2_ref[...] = y2 * dinv_ref[...]


_tc2 = pl.pallas_call(
    _tc2_body,
    grid=(_NBLK,),
    in_specs=[
        pl.BlockSpec((_BLK, H), lambda i: (i, 0)),
        pl.BlockSpec((2, _BLK, H), lambda i: (0, i, 0)),
        pl.BlockSpec((_BLK, H), lambda i: (i, 0)),
        pl.BlockSpec((1, H), lambda i: (0, 0)),
        pl.BlockSpec((H, H), lambda i: (0, 0)),
    ],
    out_specs=pl.BlockSpec((_BLK, H), lambda i: (i, 0)),
    out_shape=jax.ShapeDtypeStruct((NP, H), _f32),
)


def _tc3_body(y2_ref, r_ref, dinv_ref, b2_ref, out_ref):
    r = r_ref[...]
    acc = y2_ref[...] + r[0] + r[1]
    out_ref[...] = jnp.maximum(acc * dinv_ref[...] + b2_ref[...], 0.0)


_tc3 = pl.pallas_call(
    _tc3_body,
    grid=(_NBLK,),
    in_specs=[
        pl.BlockSpec((_BLK, H), lambda i: (i, 0)),
        pl.BlockSpec((2, _BLK, H), lambda i: (0, i, 0)),
        pl.BlockSpec((_BLK, H), lambda i: (i, 0)),
        pl.BlockSpec((1, H), lambda i: (0, 0)),
    ],
    out_specs=pl.BlockSpec((_BLK, H), lambda i: (i, 0)),
    out_shape=jax.ShapeDtypeStruct((NP, H), _f32),
)


def kernel(x, edge_index, W1, b1, W2, b2):
    src = edge_index[0]
    dst = edge_index[1]
    pad = jnp.full((EP - E,), N, dtype=jnp.int32)
    srcr = jnp.concatenate([src, pad]).reshape(NWIN_PAD, WIN)
    dstr = jnp.concatenate([dst, pad]).reshape(NWIN_PAD, WIN)
    xp = jnp.pad(x, ((0, NP - N), (0, 0)))

    degp = _deg(dstr)
    y1, dinv = _tc1(xp, W1, degp)
    q = _agg(y1, srcr, dstr)
    y2 = _tc2(y1, q, dinv, b1.reshape(1, H), W2)
    r = _agg(y2, srcr, dstr)
    outp = _tc3(y2, r, dinv, b2.reshape(1, H))
    return outp[:N]
